# Initial kernel scaffold; baseline (speedup 1.0000x reference)
import jax
import jax.numpy as jnp
from jax.experimental import pallas as pl


def kernel(coord, feat, offset, skip_coord, skip_feat, skip_offset, W_proj, b_proj, g_proj, be_proj, W_skip, b_skip, g_skip, be_skip):
    raise NotImplementedError("write your pallas kernel here")


# speedup vs baseline: 6.9524x; 6.9524x over previous
import functools

import jax
import jax.numpy as jnp
from jax import lax
from jax.experimental import pallas as pl
from jax.experimental.pallas import tpu as pltpu
from jax.experimental.pallas import tpu_sc as plsc

_N = 4096
_NS = 16384
_C = 512
_EPS = 1e-5


def _mm_stats_body(x_ref, w_ref, b_ref, y_ref, stats_ref):
    y = jnp.dot(x_ref[...], w_ref[...], preferred_element_type=jnp.float32)
    y = y + b_ref[...]
    y_ref[...] = y
    st = jnp.concatenate(
        [jnp.sum(y, axis=0, keepdims=True),
         jnp.sum(y * y, axis=0, keepdims=True)], axis=0)

    @pl.when(pl.program_id(0) == 0)
    def _init():
        stats_ref[...] = st

    @pl.when(pl.program_id(0) > 0)
    def _acc():
        stats_ref[...] = stats_ref[...] + st


def _mm_stats(x, w, b, tile):
    n = x.shape[0]
    return pl.pallas_call(
        _mm_stats_body,
        grid=(n // tile,),
        in_specs=[
            pl.BlockSpec((tile, _C), lambda i: (i, 0)),
            pl.BlockSpec((_C, _C), lambda i: (0, 0)),
            pl.BlockSpec((1, _C), lambda i: (0, 0)),
        ],
        out_specs=[
            pl.BlockSpec((tile, _C), lambda i: (i, 0)),
            pl.BlockSpec((2, _C), lambda i: (0, 0)),
        ],
        out_shape=[
            jax.ShapeDtypeStruct((n, _C), jnp.float32),
            jax.ShapeDtypeStruct((2, _C), jnp.float32),
        ],
    )(x, w, b)


def _bn_body(inv_n, add_interp, *refs):
    if add_interp:
        y_ref, interp_ref, stats_ref, g_ref, be_ref, o_ref = refs
    else:
        y_ref, stats_ref, g_ref, be_ref, o_ref = refs
    mean = stats_ref[0:1, :] * inv_n
    var = stats_ref[1:2, :] * inv_n - mean * mean
    scale = lax.rsqrt(var + _EPS) * g_ref[...]
    s = jnp.maximum((y_ref[...] - mean) * scale + be_ref[...], 0.0)
    if add_interp:
        s = s + interp_ref[...]
    o_ref[...] = s


def _bn_relu(y, stats, g, be, tile, interp=None):
    n = y.shape[0]
    add_interp = interp is not None
    in_specs = [pl.BlockSpec((tile, _C), lambda i: (i, 0))]
    args = [y]
    if add_interp:
        in_specs.append(pl.BlockSpec((tile, _C), lambda i: (i, 0)))
        args.append(interp)
    in_specs += [
        pl.BlockSpec((2, _C), lambda i: (0, 0)),
        pl.BlockSpec((1, _C), lambda i: (0, 0)),
        pl.BlockSpec((1, _C), lambda i: (0, 0)),
    ]
    args += [stats, g, be]
    return pl.pallas_call(
        functools.partial(_bn_body, 1.0 / n, add_interp),
        grid=(n // tile,),
        in_specs=in_specs,
        out_specs=pl.BlockSpec((tile, _C), lambda i: (i, 0)),
        out_shape=jax.ShapeDtypeStruct((n, _C), jnp.float32),
    )(*args)


_KTILE = 128


def _knn_body(sc_ref, ct_ref, i1_ref, i2_ref, i3_ref, w1_ref, w2_ref, w3_ref):
    s = sc_ref[...]
    ct = ct_ref[...]
    t0 = s[:, 0:1] - ct[0:1, :]
    t1 = s[:, 1:2] - ct[1:2, :]
    t2 = s[:, 2:3] - ct[2:3, :]
    d2 = t0 * t0 + t1 * t1 + t2 * t2
    j = lax.broadcasted_iota(jnp.int32, d2.shape, 1)
    big = jnp.int32(2147483647)
    inf = jnp.float32(jnp.inf)

    def min_arg(d):
        m = jnp.min(d, axis=1, keepdims=True)
        i = jnp.min(jnp.where(d == m, j, big), axis=1, keepdims=True)
        return m, i

    m1, idx1 = min_arg(d2)
    d2 = jnp.where(j == idx1, inf, d2)
    m2, idx2 = min_arg(d2)
    d2 = jnp.where(j == idx2, inf, d2)
    m3, idx3 = min_arg(d2)

    def recip(m):
        dist = jnp.sqrt(jnp.maximum(m, 1e-12))
        return 1.0 / (dist + 1e-8)

    r1 = recip(m1)
    r2 = recip(m2)
    r3 = recip(m3)
    norm = r1 + r2 + r3
    i1_ref[...] = idx1
    i2_ref[...] = idx2
    i3_ref[...] = idx3
    w1_ref[...] = r1 / norm
    w2_ref[...] = r2 / norm
    w3_ref[...] = r3 / norm


def _knn(skip_coord, ct):
    col = pl.BlockSpec((_KTILE, 1), lambda i: (i, 0))
    return pl.pallas_call(
        _knn_body,
        grid=(_NS // _KTILE,),
        in_specs=[
            pl.BlockSpec((_KTILE, 3), lambda i: (i, 0)),
            pl.BlockSpec((3, _N), lambda i: (0, 0)),
        ],
        out_specs=[col] * 6,
        out_shape=[jax.ShapeDtypeStruct((_NS, 1), jnp.int32)] * 3
        + [jax.ShapeDtypeStruct((_NS, 1), jnp.float32)] * 3,
    )(skip_coord, ct)


_NW = 32
_BW = _NS // _NW
_G = 16
_CH = _BW // _G


def _interp_sc(h, idx_flat, wb):
    mesh = plsc.VectorSubcoreMesh(core_axis_name="c", subcore_axis_name="s")

    @functools.partial(
        pl.kernel,
        out_type=jax.ShapeDtypeStruct((_NS, _C), jnp.float32),
        mesh=mesh,
        scratch_types=[
            pltpu.VMEM((3 * _G,), jnp.int32),
            pltpu.VMEM((3 * _G * 16,), jnp.float32),
            pltpu.VMEM((3 * _G, _C), jnp.float32),
            pltpu.VMEM((_G, _C), jnp.float32),
            pltpu.SemaphoreType.DMA,
        ],
    )
    def k(h_hbm, idx_hbm, wb_hbm, out_hbm, idx_v, wb_v, rows_v, out_v, sem):
        wid = lax.axis_index("s") * 2 + lax.axis_index("c")

        def chunk(ci, carry):
            pbase = wid * _BW + ci * _G
            rbase = 3 * pbase
            pltpu.sync_copy(idx_hbm.at[pl.ds(rbase, 3 * _G)], idx_v)
            pltpu.sync_copy(wb_hbm.at[pl.ds(rbase * 16, 3 * _G * 16)], wb_v)
            pltpu.async_copy(h_hbm.at[idx_v], rows_v, sem).wait()

            def point(i, c2):
                ri = 3 * i
                w0 = wb_v[pl.ds(ri * 16, 16)]
                w1 = wb_v[pl.ds(ri * 16 + 16, 16)]
                w2 = wb_v[pl.ds(ri * 16 + 32, 16)]
                for v in range(_C // 16):
                    o = v * 16
                    out_v[i, pl.ds(o, 16)] = (
                        rows_v[ri, pl.ds(o, 16)] * w0
                        + rows_v[ri + 1, pl.ds(o, 16)] * w1
                        + rows_v[ri + 2, pl.ds(o, 16)] * w2)
                return c2

            lax.fori_loop(0, _G, point, 0)
            pltpu.sync_copy(out_v, out_hbm.at[pl.ds(pbase, _G)])
            return carry

        lax.fori_loop(0, _CH, chunk, 0)

    return k(h, idx_flat, wb)


def kernel(coord, feat, offset, skip_coord, skip_feat, skip_offset,
           W_proj, b_proj, g_proj, be_proj, W_skip, b_skip, g_skip, be_skip):
    b_p = b_proj.reshape(1, _C)
    g_p = g_proj.reshape(1, _C)
    be_p = be_proj.reshape(1, _C)
    b_s = b_skip.reshape(1, _C)
    g_s = g_skip.reshape(1, _C)
    be_s = be_skip.reshape(1, _C)

    y_p, st_p = _mm_stats(feat, W_proj, b_p, 512)
    h = _bn_relu(y_p, st_p, g_p, be_p, 512)

    i1, i2, i3, w1, w2, w3 = _knn(skip_coord, coord.T)
    idx_flat = jnp.concatenate([i1, i2, i3], axis=1).reshape(-1)
    w = jnp.concatenate([w1, w2, w3], axis=1)
    wb = jnp.broadcast_to(w.reshape(_NS * 3, 1), (_NS * 3, 16)).reshape(-1)

    interp = _interp_sc(h, idx_flat, wb)

    y_s, st_s = _mm_stats(skip_feat, W_skip, b_s, 512)
    out_feat = _bn_relu(y_s, st_s, g_s, be_s, 512, interp=interp)

    return (skip_coord, out_feat, skip_offset)
